# Initial kernel scaffold; baseline (speedup 1.0000x reference)
import jax
import jax.numpy as jnp
from jax.experimental import pallas as pl


def kernel(x, edge_index, W1, b1, W2, b2):
    raise NotImplementedError("write your pallas kernel here")


# speedup vs baseline: 12.7343x; 12.7343x over previous
import functools

import jax
import jax.numpy as jnp
from jax import lax
from jax.experimental import pallas as pl
from jax.experimental.pallas import tpu as pltpu
from jax.experimental.pallas import tpu_sc as plsc

NC = 2
NS = 16
NW = NC * NS
CH = 128


def _round_up(a, b):
    return (a + b - 1) // b * b


def _sc_degree(dst_p, n_deg, n_chunk):
    mesh = plsc.VectorSubcoreMesh(core_axis_name="c", subcore_axis_name="s")
    zt = n_deg // NS
    ew = n_chunk * CH

    @functools.partial(
        pl.kernel,
        out_type=jax.ShapeDtypeStruct((NC * n_deg,), jnp.float32),
        mesh=mesh,
        scratch_types=[
            pltpu.VMEM((CH,), jnp.int32),
            pltpu.VMEM((CH,), jnp.float32),
            pltpu.VMEM((zt,), jnp.float32),
            pltpu.VMEM_SHARED((n_deg,), jnp.float32),
        ],
    )
    def k(dst_h, out_h, dst_v, ones_v, stage_v, deg_sh):
        c = lax.axis_index("c")
        s = lax.axis_index("s")
        wid = s * NC + c
        for j in range(CH // 16):
            ones_v[pl.ds(16 * j, 16)] = jnp.ones((16,), jnp.float32)

        def zbody(i, carry):
            stage_v[pl.ds(i * 16, 16)] = jnp.zeros((16,), jnp.float32)
            return carry

        lax.fori_loop(0, zt // 16, zbody, 0)
        pltpu.sync_copy(stage_v, deg_sh.at[pl.ds(s * zt, zt)])
        plsc.subcore_barrier()
        base0 = wid * ew

        def body(i, carry):
            b = base0 + i * CH
            pltpu.sync_copy(dst_h.at[pl.ds(b, CH)], dst_v)
            pltpu.sync_copy(ones_v, deg_sh.at[dst_v], add=True)
            return carry

        lax.fori_loop(0, n_chunk, body, 0)
        plsc.subcore_barrier()
        pltpu.sync_copy(deg_sh.at[pl.ds(s * zt, zt)], stage_v)
        pltpu.sync_copy(stage_v, out_h.at[pl.ds(c * n_deg + s * zt, zt)])

    return k(dst_p).reshape(NC, n_deg)


def _sc_aggregate(table, src_p, dst_p, n, n_acc, n_chunk):
    d = table.shape[1]
    mesh = plsc.VectorSubcoreMesh(core_axis_name="c", subcore_axis_name="s")
    zt = n_acc // NS
    ew = n_chunk * CH

    @functools.partial(
        pl.kernel,
        out_type=jax.ShapeDtypeStruct((NC, n_acc, d), jnp.float32),
        mesh=mesh,
        scratch_types=[
            pltpu.VMEM((CH,), jnp.int32),
            pltpu.VMEM((CH,), jnp.int32),
            pltpu.VMEM((CH, d), jnp.float32),
            pltpu.VMEM_SHARED((n_acc, d), jnp.float32),
            pltpu.SemaphoreType.DMA,
        ],
    )
    def k(tab_h, src_h, dst_h, out_h, src_v, dst_v, rows_v, acc_sh, sem):
        c = lax.axis_index("c")
        s = lax.axis_index("s")
        wid = s * NC + c

        def zrow(i, carry):
            for j in range(d // 16):
                rows_v[i, pl.ds(j * 16, 16)] = jnp.zeros((16,), jnp.float32)
            return carry

        lax.fori_loop(0, CH, zrow, 0)

        def zcopy(i, carry):
            pltpu.sync_copy(rows_v, acc_sh.at[pl.ds(s * zt + i * CH, CH)])
            return carry

        lax.fori_loop(0, zt // CH, zcopy, 0)
        plsc.subcore_barrier()
        base0 = wid * ew

        def body(i, carry):
            b = base0 + i * CH
            pltpu.sync_copy(src_h.at[pl.ds(b, CH)], src_v)
            pltpu.sync_copy(dst_h.at[pl.ds(b, CH)], dst_v)
            pltpu.async_copy(tab_h.at[src_v], rows_v, sem).wait()
            pltpu.sync_copy(rows_v, acc_sh.at[dst_v], add=True)
            return carry

        lax.fori_loop(0, n_chunk, body, 0)
        plsc.subcore_barrier()

        def ocopy(i, carry):
            r0 = s * zt + i * CH
            pltpu.sync_copy(acc_sh.at[pl.ds(r0, CH)], rows_v)
            pltpu.sync_copy(rows_v, out_h.at[c, pl.ds(r0, CH)])
            return carry

        lax.fori_loop(0, zt // CH, ocopy, 0)

    return k(table, src_p, dst_p)


def _c1_body(x_ref, dt_ref, xs_ref, disb_ref):
    dt = dt_ref[...]
    deg = dt[:, 0:1] + dt[:, 1:2] + 1.0
    dis = lax.rsqrt(deg)
    disb = jnp.broadcast_to(dis, x_ref.shape)
    disb_ref[...] = disb
    xs_ref[...] = x_ref[...] * disb


def _tc_scale(x, degt):
    n, d = x.shape
    r = 2000
    g = n // r
    return pl.pallas_call(
        _c1_body,
        grid=(g,),
        in_specs=[
            pl.BlockSpec((r, d), lambda i: (i, 0)),
            pl.BlockSpec((r, NC), lambda i: (i, 0)),
        ],
        out_specs=[
            pl.BlockSpec((r, d), lambda i: (i, 0)),
            pl.BlockSpec((r, d), lambda i: (i, 0)),
        ],
        out_shape=[
            jax.ShapeDtypeStruct((n, d), jnp.float32),
            jax.ShapeDtypeStruct((n, d), jnp.float32),
        ],
    )(x, degt)


def _c2_body(z_ref, xs_ref, disb_ref, w1_ref, b1_ref, w2_ref, o_ref):
    disb = disb_ref[...]
    a = disb * (z_ref[0] + z_ref[1] + xs_ref[...])
    h = jnp.dot(a, w1_ref[...], preferred_element_type=jnp.float32,
                precision=lax.Precision.HIGHEST)
    h = jnp.maximum(h + b1_ref[...], 0.0)
    t = jnp.dot(h, w2_ref[...], preferred_element_type=jnp.float32,
                precision=lax.Precision.HIGHEST)
    o_ref[...] = disb * t


def _tc_layer(z, xs, disb, W1, b1, W2):
    n, d = xs.shape
    h = W1.shape[1]
    r = 2000
    g = n // r
    return pl.pallas_call(
        _c2_body,
        grid=(g,),
        in_specs=[
            pl.BlockSpec((NC, r, d), lambda i: (0, i, 0)),
            pl.BlockSpec((r, d), lambda i: (i, 0)),
            pl.BlockSpec((r, d), lambda i: (i, 0)),
            pl.BlockSpec((d, h), lambda i: (0, 0)),
            pl.BlockSpec((1, h), lambda i: (0, 0)),
            pl.BlockSpec((h, d), lambda i: (0, 0)),
        ],
        out_specs=pl.BlockSpec((r, d), lambda i: (i, 0)),
        out_shape=jax.ShapeDtypeStruct((n, d), jnp.float32),
    )(z, xs, disb, W1, b1, W2)


def _c3_body(u_ref, ts_ref, disb_ref, b2_ref, o_ref):
    o_ref[...] = disb_ref[...] * (u_ref[0] + u_ref[1] + ts_ref[...]) + b2_ref[...]


def _tc_out(u, ts, disb, b2):
    n, d = ts.shape
    r = 2000
    g = n // r
    return pl.pallas_call(
        _c3_body,
        grid=(g,),
        in_specs=[
            pl.BlockSpec((NC, r, d), lambda i: (0, i, 0)),
            pl.BlockSpec((r, d), lambda i: (i, 0)),
            pl.BlockSpec((r, d), lambda i: (i, 0)),
            pl.BlockSpec((1, d), lambda i: (0, 0)),
        ],
        out_specs=pl.BlockSpec((r, d), lambda i: (i, 0)),
        out_shape=jax.ShapeDtypeStruct((n, d), jnp.float32),
    )(u, ts, disb, b2)


def kernel(x, edge_index, W1, b1, W2, b2):
    n, d = x.shape
    e = edge_index.shape[1]

    ew = _round_up(e, NW * CH) // NW
    n_chunk = ew // CH
    e_pad = ew * NW
    n_deg = _round_up(n + 1, NS * CH)
    n_acc = _round_up(n + 1, NS * CH)

    src = edge_index[0].astype(jnp.int32)
    dst = edge_index[1].astype(jnp.int32)
    pad = e_pad - e
    src_p = jnp.concatenate([src, jnp.zeros((pad,), jnp.int32)])
    dst_p = jnp.concatenate([dst, jnp.full((pad,), n, jnp.int32)])

    degp = _sc_degree(dst_p, n_deg, n_chunk)
    degt = degp.T[:n]

    xs, disb = _tc_scale(x, degt)
    z = _sc_aggregate(xs, src_p, dst_p, n, n_acc, n_chunk)
    ts = _tc_layer(z, xs, disb, W1, b1.reshape(1, -1), W2)
    u = _sc_aggregate(ts, src_p, dst_p, n, n_acc, n_chunk)
    out = _tc_out(u, ts, disb, b2.reshape(1, -1))
    return out
